# baseline (speedup 1.0000x reference)
import functools

import jax
import jax.numpy as jnp
from jax import lax
from jax.experimental import pallas as pl
from jax.experimental.pallas import tpu as pltpu
from jax.experimental.pallas import tpu_sc as plsc

N = 10000
NP = 10240
E = 320000
D = 128
CW = 16
NC = 2
NS = 16
NW = NC * NS
EPW = E // NW
C = 80
NCHUNK = EPW // C
RPS = NP // NS
NBUF = 4

_mesh = plsc.VectorSubcoreMesh(core_axis_name="c", subcore_axis_name="s")


@functools.partial(
    pl.kernel,
    out_type=jax.ShapeDtypeStruct((NC, NP, CW), jnp.float32),
    mesh=_mesh,
    scratch_types=[
        pltpu.VMEM((C, CW), jnp.float32),
        pltpu.VMEM((NCHUNK, C), jnp.int32),
        pltpu.SemaphoreType.DMA,
        pltpu.VMEM_SHARED((NP, CW), jnp.float32),
    ],
)
def _sc_count(dst3_hbm, zeros_hbm, cnt_hbm, ones_v, didx_v, sem, table_sh):
    c = lax.axis_index("c")
    s = lax.axis_index("s")
    wid = c * NS + s

    @pl.loop(0, C)
    def _(r):
        ones_v[r, :] = jnp.full((CW,), 1.0, jnp.float32)

    pltpu.sync_copy(dst3_hbm.at[wid], didx_v)
    row0 = s * RPS
    pltpu.sync_copy(zeros_hbm.at[pl.ds(row0, RPS)], table_sh.at[pl.ds(row0, RPS)])
    plsc.subcore_barrier()

    @pl.loop(0, NCHUNK)
    def _(j):
        pltpu.async_copy(ones_v, table_sh.at[didx_v.at[j]], sem, add=True)

        @pl.when(j >= NBUF)
        def _():
            pltpu.make_async_copy(
                ones_v, table_sh.at[didx_v.at[j - NBUF]], sem).wait()

    for jt in range(NCHUNK - NBUF, NCHUNK):
        pltpu.make_async_copy(ones_v, table_sh.at[didx_v.at[jt]], sem).wait()

    plsc.subcore_barrier()
    pltpu.sync_copy(table_sh.at[pl.ds(row0, RPS)], cnt_hbm.at[c, pl.ds(row0, RPS)])


NIB = 6
UNROLL = 12


@functools.partial(
    pl.kernel,
    out_type=jax.ShapeDtypeStruct((NC, NP, D), jnp.float32),
    mesh=_mesh,
    scratch_types=[
        pltpu.VMEM((NIB * 8, C), jnp.int32),
        pltpu.VMEM((NBUF, C, D), jnp.float32),
        [pltpu.SemaphoreType.DMA] * NIB,
        [pltpu.SemaphoreType.DMA] * NBUF,
        [pltpu.SemaphoreType.DMA] * NBUF,
        pltpu.VMEM_SHARED((NP, D), jnp.float32),
    ],
)
def _sc_agg(y_hbm, ei4_hbm, p_hbm, idx_v, rows_v, isems, gs, ss, acc_sh):
    c = lax.axis_index("c")
    s = lax.axis_index("s")
    wid = c * NS + s

    row0 = s * RPS
    pltpu.sync_copy(y_hbm.at[pl.ds(row0, RPS)], acc_sh.at[pl.ds(row0, RPS)])
    plsc.subcore_barrier()

    def idx_load(k, ib):
        pltpu.async_copy(ei4_hbm.at[wid, k], idx_v.at[pl.ds(8 * ib, 2)], isems[ib])

    def idx_wait(k, ib):
        pltpu.make_async_copy(
            ei4_hbm.at[wid, k], idx_v.at[pl.ds(8 * ib, 2)], isems[ib]).wait()

    def gather(b, ib):
        pltpu.async_copy(y_hbm.at[idx_v.at[8 * ib]], rows_v.at[b], gs[b])

    def wait_gather(b, ib):
        pltpu.make_async_copy(
            y_hbm.at[idx_v.at[8 * ib]], rows_v.at[b], gs[b]).wait()

    def scatter(b, ib):
        pltpu.async_copy(rows_v.at[b], acc_sh.at[idx_v.at[8 * ib + 1]], ss[b],
                         add=True)

    def wait_scatter(b, ib):
        pltpu.make_async_copy(
            rows_v.at[b], acc_sh.at[idx_v.at[8 * ib + 1]], ss[b]).wait()

    def visit(k, b, ib, in_loop):
        nb = (b + 2) % NBUF
        pib = (ib + 4) % NIB
        if in_loop:
            pl.when(k >= 2)(lambda: wait_scatter(nb, pib))
        else:
            wait_scatter(nb, pib)
        if in_loop or k + 4 < NCHUNK:
            idx_load(k + 4, pib)
        if in_loop or k + 2 < NCHUNK:
            idx_wait(k + 2, (ib + 2) % NIB)
            gather((b + 2) % NBUF, (ib + 2) % NIB)
        wait_gather(b, ib)
        scatter(b, ib)

    for k0 in range(4):
        idx_load(k0, k0)
    idx_wait(0, 0)
    gather(0, 0)
    idx_wait(1, 1)
    gather(1, 1)

    @pl.loop(0, NCHUNK - 5, step=UNROLL)
    def _(j):
        for off in range(UNROLL):
            visit(j + off, off % NBUF, off % NIB, True)

    for k in range(NCHUNK - 5, NCHUNK):
        visit(k, k % NBUF, k % NIB, False)

    wait_scatter((NCHUNK - 2) % NBUF, (NCHUNK - 2) % NIB)
    wait_scatter((NCHUNK - 1) % NBUF, (NCHUNK - 1) % NIB)

    plsc.subcore_barrier()
    pltpu.sync_copy(acc_sh.at[pl.ds(row0, RPS)], p_hbm.at[c, pl.ds(row0, RPS)])


_BM = 1000


def _mm_body(x_ref, w_ref, xw_ref):
    xw_ref[...] = lax.dot_general(
        x_ref[...], w_ref[...], (((1,), (0,)), ((), ())),
        precision=lax.Precision.HIGHEST, preferred_element_type=jnp.float32)


def _tc_matmul(x, w):
    return pl.pallas_call(
        _mm_body,
        grid=(N // _BM,),
        in_specs=[
            pl.BlockSpec((_BM, D), lambda i: (i, 0)),
            pl.BlockSpec((D, D), lambda i: (0, 0)),
        ],
        out_specs=pl.BlockSpec((_BM, D), lambda i: (i, 0)),
        out_shape=jax.ShapeDtypeStruct((N, D), jnp.float32),
    )(x, w)


def _scale_body(xw_ref, cnt_ref, y_ref):
    deg = cnt_ref[0, :, 0] + cnt_ref[1, :, 0] + 1.0
    dis = lax.rsqrt(deg)
    y_ref[...] = xw_ref[...] * dis[:, None]


def _tc_scale(xw, cnt):
    return pl.pallas_call(
        _scale_body,
        grid=(N // _BM,),
        in_specs=[
            pl.BlockSpec((_BM, D), lambda i: (i, 0)),
            pl.BlockSpec((NC, _BM, CW), lambda i: (0, i, 0)),
        ],
        out_specs=pl.BlockSpec((_BM, D), lambda i: (i, 0)),
        out_shape=jax.ShapeDtypeStruct((N, D), jnp.float32),
    )(xw, cnt)


def _comb_body(p_ref, y_ref, cnt_ref, b_ref, o_ref):
    deg = cnt_ref[0, :, 0] + cnt_ref[1, :, 0] + 1.0
    dis = lax.rsqrt(deg)
    o_ref[...] = dis[:, None] * (p_ref[0] + p_ref[1] - y_ref[...]) + b_ref[...]


def _tc_combine(p, y, cnt, b2d):
    return pl.pallas_call(
        _comb_body,
        grid=(N // _BM,),
        in_specs=[
            pl.BlockSpec((NC, _BM, D), lambda i: (0, i, 0)),
            pl.BlockSpec((_BM, D), lambda i: (i, 0)),
            pl.BlockSpec((NC, _BM, CW), lambda i: (0, i, 0)),
            pl.BlockSpec((1, D), lambda i: (0, 0)),
        ],
        out_specs=pl.BlockSpec((_BM, D), lambda i: (i, 0)),
        out_shape=jax.ShapeDtypeStruct((N, D), jnp.float32),
    )(p, y, cnt, b2d)


def kernel(x, edge_index, W, b):
    src3 = edge_index[0].astype(jnp.int32).reshape(NW, NCHUNK, C)
    dst3 = edge_index[1].astype(jnp.int32).reshape(NW, NCHUNK, C)
    ei4 = jnp.stack([src3, dst3], axis=2)
    zeros16 = jnp.zeros((NP, CW), jnp.float32)
    cnt = _sc_count(dst3, zeros16)
    xw = _tc_matmul(x, W)
    y = _tc_scale(xw, cnt)
    yp = jnp.concatenate([y, jnp.zeros((NP - N, D), jnp.float32)], axis=0)
    p = _sc_agg(yp, ei4)
    return _tc_combine(p, y, cnt, b.reshape(1, D))


# speedup vs baseline: 1.0540x; 1.0540x over previous
import functools

import jax
import jax.numpy as jnp
from jax import lax
from jax.experimental import pallas as pl
from jax.experimental.pallas import tpu as pltpu
from jax.experimental.pallas import tpu_sc as plsc

N = 10000
NP = 10240
E = 320000
D = 128
CW = 16
NC = 2
NS = 16
NW = NC * NS
EPW = E // NW
C = 80
NCHUNK = EPW // C
RPS = NP // NS
NBUF = 4

_mesh = plsc.VectorSubcoreMesh(core_axis_name="c", subcore_axis_name="s")


@functools.partial(
    pl.kernel,
    out_type=jax.ShapeDtypeStruct((NC, NP, CW), jnp.float32),
    mesh=_mesh,
    scratch_types=[
        pltpu.VMEM((C, CW), jnp.float32),
        pltpu.VMEM((NCHUNK, C), jnp.int32),
        pltpu.SemaphoreType.DMA,
        pltpu.VMEM_SHARED((NP, CW), jnp.float32),
    ],
)
def _sc_count(dst3_hbm, zeros_hbm, cnt_hbm, ones_v, didx_v, sem, table_sh):
    c = lax.axis_index("c")
    s = lax.axis_index("s")
    wid = c * NS + s

    @pl.loop(0, C)
    def _(r):
        ones_v[r, :] = jnp.full((CW,), 1.0, jnp.float32)

    pltpu.sync_copy(dst3_hbm.at[wid], didx_v)
    row0 = s * RPS
    pltpu.sync_copy(zeros_hbm.at[pl.ds(row0, RPS)], table_sh.at[pl.ds(row0, RPS)])
    plsc.subcore_barrier()

    @pl.loop(0, NCHUNK)
    def _(j):
        pltpu.async_copy(ones_v, table_sh.at[didx_v.at[j]], sem, add=True)

        @pl.when(j >= NBUF)
        def _():
            pltpu.make_async_copy(
                ones_v, table_sh.at[didx_v.at[j - NBUF]], sem).wait()

    for jt in range(NCHUNK - NBUF, NCHUNK):
        pltpu.make_async_copy(ones_v, table_sh.at[didx_v.at[jt]], sem).wait()

    plsc.subcore_barrier()
    pltpu.sync_copy(table_sh.at[pl.ds(row0, RPS)], cnt_hbm.at[c, pl.ds(row0, RPS)])


NIB = 6
UNROLL = 12


@functools.partial(
    pl.kernel,
    out_type=jax.ShapeDtypeStruct((NC, NP, D), jnp.float32),
    mesh=_mesh,
    scratch_types=[
        pltpu.VMEM((NIB * 8, C), jnp.int32),
        pltpu.VMEM((NBUF, C, D), jnp.float32),
        [pltpu.SemaphoreType.DMA] * NIB,
        [pltpu.SemaphoreType.DMA] * NBUF,
        [pltpu.SemaphoreType.DMA] * NBUF,
        pltpu.VMEM_SHARED((NP, D), jnp.float32),
    ],
)
def _sc_agg(y_hbm, ei4_hbm, p_hbm, idx_v, rows_v, isems, gs, ss, acc_sh):
    c = lax.axis_index("c")
    s = lax.axis_index("s")
    wid = c * NS + s

    row0 = s * RPS
    pltpu.sync_copy(y_hbm.at[pl.ds(row0, RPS)], acc_sh.at[pl.ds(row0, RPS)])
    plsc.subcore_barrier()

    def idx_load(k, ib):
        pltpu.async_copy(ei4_hbm.at[wid, k], idx_v.at[pl.ds(8 * ib, 2)], isems[ib])

    def idx_wait(k, ib):
        pltpu.make_async_copy(
            ei4_hbm.at[wid, k], idx_v.at[pl.ds(8 * ib, 2)], isems[ib]).wait()

    def gather(b, ib):
        pltpu.async_copy(y_hbm.at[idx_v.at[8 * ib]], rows_v.at[b], gs[b])

    def wait_gather(b, ib):
        pltpu.make_async_copy(
            y_hbm.at[idx_v.at[8 * ib]], rows_v.at[b], gs[b]).wait()

    def scatter(b, ib):
        pltpu.async_copy(rows_v.at[b], acc_sh.at[idx_v.at[8 * ib + 1]], ss[b],
                         add=True)

    def wait_scatter(b, ib):
        pltpu.make_async_copy(
            rows_v.at[b], acc_sh.at[idx_v.at[8 * ib + 1]], ss[b]).wait()

    def visit(k, b, ib, in_loop):
        nb = (b + 2) % NBUF
        pib = (ib + 4) % NIB
        if in_loop:
            pl.when(k >= 2)(lambda: wait_scatter(nb, pib))
        else:
            wait_scatter(nb, pib)
        if in_loop or k + 4 < NCHUNK:
            idx_load(k + 4, pib)
        if in_loop or k + 2 < NCHUNK:
            idx_wait(k + 2, (ib + 2) % NIB)
            gather((b + 2) % NBUF, (ib + 2) % NIB)
        wait_gather(b, ib)
        scatter(b, ib)

    for k0 in range(4):
        idx_load(k0, k0)
    idx_wait(0, 0)
    gather(0, 0)
    idx_wait(1, 1)
    gather(1, 1)

    @pl.loop(0, NCHUNK - 5, step=UNROLL)
    def _(j):
        for off in range(UNROLL):
            visit(j + off, off % NBUF, off % NIB, True)

    for k in range(NCHUNK - 5, NCHUNK):
        visit(k, k % NBUF, k % NIB, False)

    wait_scatter((NCHUNK - 2) % NBUF, (NCHUNK - 2) % NIB)
    wait_scatter((NCHUNK - 1) % NBUF, (NCHUNK - 1) % NIB)

    plsc.subcore_barrier()
    pltpu.sync_copy(acc_sh.at[pl.ds(row0, RPS)], p_hbm.at[c, pl.ds(row0, RPS)])


_BM = 1000


def _mm_body(x_ref, w_ref, cnt_ref, y_ref):
    deg = cnt_ref[0, :, 0] + cnt_ref[1, :, 0] + 1.0
    dis = lax.rsqrt(deg)
    xw = lax.dot_general(
        x_ref[...], w_ref[...], (((1,), (0,)), ((), ())),
        precision=lax.Precision.HIGHEST, preferred_element_type=jnp.float32)
    y_ref[...] = xw * dis[:, None]


def _tc_matmul_scale(x, w, cnt):
    return pl.pallas_call(
        _mm_body,
        grid=(N // _BM,),
        in_specs=[
            pl.BlockSpec((_BM, D), lambda i: (i, 0)),
            pl.BlockSpec((D, D), lambda i: (0, 0)),
            pl.BlockSpec((NC, _BM, CW), lambda i: (0, i, 0)),
        ],
        out_specs=pl.BlockSpec((_BM, D), lambda i: (i, 0)),
        out_shape=jax.ShapeDtypeStruct((NP, D), jnp.float32),
    )(x, w, cnt)


def _comb_body(p_ref, y_ref, cnt_ref, b_ref, o_ref):
    deg = cnt_ref[0, :, 0] + cnt_ref[1, :, 0] + 1.0
    dis = lax.rsqrt(deg)
    o_ref[...] = dis[:, None] * (p_ref[0] + p_ref[1] - y_ref[...]) + b_ref[...]


def _tc_combine(p, y, cnt, b2d):
    return pl.pallas_call(
        _comb_body,
        grid=(N // _BM,),
        in_specs=[
            pl.BlockSpec((NC, _BM, D), lambda i: (0, i, 0)),
            pl.BlockSpec((_BM, D), lambda i: (i, 0)),
            pl.BlockSpec((NC, _BM, CW), lambda i: (0, i, 0)),
            pl.BlockSpec((1, D), lambda i: (0, 0)),
        ],
        out_specs=pl.BlockSpec((_BM, D), lambda i: (i, 0)),
        out_shape=jax.ShapeDtypeStruct((N, D), jnp.float32),
    )(p, y, cnt, b2d)


def kernel(x, edge_index, W, b):
    src3 = edge_index[0].astype(jnp.int32).reshape(NW, NCHUNK, C)
    dst3 = edge_index[1].astype(jnp.int32).reshape(NW, NCHUNK, C)
    ei4 = jnp.stack([src3, dst3], axis=2)
    zeros16 = jnp.zeros((NP, CW), jnp.float32)
    cnt = _sc_count(dst3, zeros16)
    y = _tc_matmul_scale(x, W, cnt)
    p = _sc_agg(y, ei4)
    return _tc_combine(p, y, cnt, b.reshape(1, D))
